# baseline (speedup 1.0000x reference)
import functools

import jax
import jax.numpy as jnp
from jax import lax
from jax.experimental import pallas as pl
from jax.experimental.pallas import tpu as pltpu
from jax.experimental.pallas import tpu_sc as plsc

NC = 2
NS = 16
NW = NC * NS

C = 128
S = 4
L = 200
BB = 32
CPW = (L * BB) // NW


def _gather_kernel(d):
    dg = d // 8
    mesh = plsc.VectorSubcoreMesh(
        core_axis_name="c", subcore_axis_name="s",
        num_cores=NC, num_subcores=NS)

    @functools.partial(
        pl.kernel,
        out_type=jax.ShapeDtypeStruct((L, dg, BB, 8, C), jnp.float32),
        mesh=mesh,
        compiler_params=pltpu.CompilerParams(
            use_tc_tiling_on_sc=False, needs_layout_passes=False),
        scratch_types=[
            pltpu.VMEM((CPW, C), jnp.int32),
            pltpu.VMEM((S, C, d), jnp.float32),
            pltpu.VMEM((S, dg, 8, C), jnp.float32),
            [pltpu.SemaphoreType.DMA] * S,
            [pltpu.SemaphoreType.DMA] * S,
        ],
    )
    def body(table_hbm, idx_hbm, out_hbm, idx_v, rows_v, rowst_v, gsems, ssems):
        wid = lax.axis_index("s") * NC + lax.axis_index("c")
        cbase = wid * CPW

        pltpu.sync_copy(idx_hbm.at[wid], idx_v)

        rvecs = [b0 + lax.iota(jnp.int32, 16) for b0 in range(0, C, 16)]

        def gather_start(s, j):
            pltpu.async_copy(table_hbm.at[idx_v.at[j]], rows_v.at[s], gsems[s])

        def gather_wait(s):
            pltpu.make_async_copy(
                table_hbm.at[idx_v.at[0]], rows_v.at[s], gsems[s]).wait()

        def store_start(s, j):
            c = cbase + j
            pltpu.async_copy(
                rowst_v.at[s], out_hbm.at[c // BB, :, c % BB], ssems[s])

        def store_wait(s):
            pltpu.make_async_copy(
                rowst_v.at[s], out_hbm.at[0, :, 0], ssems[s]).wait()

        def transpose(s):
            @pl.loop(0, d, unroll=4)
            def _(dd):
                dvec = jnp.full((16,), dd, dtype=jnp.int32)
                di, dj = dd // 8, dd % 8
                for g in range(C // 16):
                    val = plsc.load_gather(rows_v.at[s], [rvecs[g], dvec])
                    rowst_v[s, di, dj, pl.ds(g * 16, 16)] = val

        for s in range(S):
            gather_start(s, s)

        for s in range(S):
            gather_wait(s)
            transpose(s)
            store_start(s, s)
            gather_start(s, s + S)

        @pl.loop(S, CPW - S, step=S)
        def _(j0):
            for s in range(S):
                j = j0 + s
                gather_wait(s)
                store_wait(s)
                transpose(s)
                store_start(s, j)
                gather_start(s, j + S)

        for s in range(S):
            gather_wait(s)
            store_wait(s)
            transpose(s)
            store_start(s, CPW - S + s)
        for s in range(S):
            store_wait(s)

    return body


def kernel(x, table):
    batch, hist = x.shape
    vocab, d = table.shape
    idx = (
        x.T.astype(jnp.int32)
        .reshape(hist, batch // C, C)
        .reshape(NW, CPW, C)
    )
    out = _gather_kernel(d)(table, idx)
    return out.transpose(2, 4, 0, 1, 3).reshape(batch, hist, d)


# speedup vs baseline: 1.4584x; 1.4584x over previous
import functools

import jax
import jax.numpy as jnp
from jax import lax
from jax.experimental import pallas as pl
from jax.experimental.pallas import tpu as pltpu
from jax.experimental.pallas import tpu_sc as plsc

NC = 2
NS = 16
NW = NC * NS

C = 128
S = 4
L = 200
BB = 32
CPW = (L * BB) // NW


def _gather_kernel(d):
    dg = d // 8
    mesh = plsc.VectorSubcoreMesh(
        core_axis_name="c", subcore_axis_name="s",
        num_cores=NC, num_subcores=NS)

    @functools.partial(
        pl.kernel,
        out_type=jax.ShapeDtypeStruct((L, dg, BB, 8, C), jnp.float32),
        mesh=mesh,
        compiler_params=pltpu.CompilerParams(
            use_tc_tiling_on_sc=False, needs_layout_passes=False),
        scratch_types=[
            pltpu.VMEM((CPW, C), jnp.int32),
            pltpu.VMEM((S, C, d), jnp.float32),
            pltpu.VMEM((S, dg, 8, C), jnp.float32),
            [pltpu.SemaphoreType.DMA] * S,
            [pltpu.SemaphoreType.DMA] * S,
        ],
    )
    def body(table_hbm, idx_hbm, out_hbm, idx_v, rows_v, rowst_v, gsems, ssems):
        wid = lax.axis_index("s") * NC + lax.axis_index("c")
        cbase = wid * CPW

        pltpu.sync_copy(idx_hbm.at[wid], idx_v)

        rvecs = [b0 + lax.iota(jnp.int32, 16) for b0 in range(0, C, 16)]

        def gather_start(s, j):
            pltpu.async_copy(table_hbm.at[idx_v.at[j]], rows_v.at[s], gsems[s])

        def gather_wait(s):
            pltpu.make_async_copy(
                table_hbm.at[idx_v.at[0]], rows_v.at[s], gsems[s]).wait()

        def store_start(s, j):
            c = cbase + j
            pltpu.async_copy(
                rowst_v.at[s], out_hbm.at[c // BB, :, c % BB], ssems[s])

        def store_wait(s):
            pltpu.make_async_copy(
                rowst_v.at[s], out_hbm.at[0, :, 0], ssems[s]).wait()

        def transpose(s):
            @plsc.parallel_loop(0, d, unroll=4)
            def _(dd):
                dvec = jnp.full((16,), dd, dtype=jnp.int32)
                di, dj = dd // 8, dd % 8
                for g in range(C // 16):
                    val = plsc.load_gather(rows_v.at[s], [rvecs[g], dvec])
                    rowst_v[s, di, dj, pl.ds(g * 16, 16)] = val

        for s in range(S):
            gather_start(s, s)

        for s in range(S):
            gather_wait(s)
            transpose(s)
            store_start(s, s)
            gather_start(s, s + S)

        @pl.loop(S, CPW - S, step=S)
        def _(j0):
            for s in range(S):
                j = j0 + s
                gather_wait(s)
                store_wait(s)
                transpose(s)
                store_start(s, j)
                gather_start(s, j + S)

        for s in range(S):
            gather_wait(s)
            store_wait(s)
            transpose(s)
            store_start(s, CPW - S + s)
        for s in range(S):
            store_wait(s)

    return body


def kernel(x, table):
    batch, hist = x.shape
    vocab, d = table.shape
    idx = (
        x.T.astype(jnp.int32)
        .reshape(hist, batch // C, C)
        .reshape(NW, CPW, C)
    )
    out = _gather_kernel(d)(table, idx)
    return out.transpose(2, 4, 0, 1, 3).reshape(batch, hist, d)


# speedup vs baseline: 2.4192x; 1.6588x over previous
import functools

import jax
import jax.numpy as jnp
from jax import lax
from jax.experimental import pallas as pl
from jax.experimental.pallas import tpu as pltpu
from jax.experimental.pallas import tpu_sc as plsc

NC = 2
NS = 16
NW = NC * NS

C = 128
S = 4
L = 200
BB = 32
CPW = (L * BB) // NW


def _gather_kernel(d):
    dg = d // 8
    mesh = plsc.VectorSubcoreMesh(
        core_axis_name="c", subcore_axis_name="s",
        num_cores=NC, num_subcores=NS)

    @functools.partial(
        pl.kernel,
        out_type=jax.ShapeDtypeStruct((L, dg, BB, 8, C), jnp.float32),
        mesh=mesh,
        compiler_params=pltpu.CompilerParams(
            use_tc_tiling_on_sc=False, needs_layout_passes=False),
        scratch_types=[
            pltpu.VMEM((CPW, C), jnp.int32),
            pltpu.VMEM((S, C, d), jnp.float32),
            pltpu.VMEM((S, dg, 8, C + 1), jnp.float32),
            [pltpu.SemaphoreType.DMA] * S,
            [pltpu.SemaphoreType.DMA] * S,
        ],
    )
    def body(table_hbm, idx_hbm, out_hbm, idx_v, rows_v, rowst_v, gsems, ssems):
        wid = lax.axis_index("s") * NC + lax.axis_index("c")
        cbase = wid * CPW

        pltpu.sync_copy(idx_hbm.at[wid], idx_v)

        dvecs = [g * 16 + lax.iota(jnp.int32, 16) for g in range(d // 16)]
        divecs = [v // 8 for v in dvecs]
        djvecs = [v % 8 for v in dvecs]

        def gather_start(s, j):
            pltpu.async_copy(table_hbm.at[idx_v.at[j]], rows_v.at[s], gsems[s])

        def gather_wait(s):
            pltpu.make_async_copy(
                table_hbm.at[idx_v.at[0]], rows_v.at[s], gsems[s]).wait()

        def _st_src(s):
            return rowst_v.at[s, :, :, pl.ds(0, C)]

        def store_start(s, j):
            c = cbase + j
            pltpu.async_copy(
                _st_src(s), out_hbm.at[c // BB, :, c % BB], ssems[s])

        def store_wait(s):
            pltpu.make_async_copy(
                _st_src(s), out_hbm.at[0, :, 0], ssems[s]).wait()

        def transpose(s):
            @plsc.parallel_loop(0, C, unroll=2)
            def _(b):
                bvec = jnp.full((16,), b, dtype=jnp.int32)
                for g in range(d // 16):
                    val = rows_v[s, b, pl.ds(g * 16, 16)]
                    plsc.store_scatter(
                        rowst_v.at[s], [divecs[g], djvecs[g], bvec], val)

        for s in range(S):
            gather_start(s, s)

        for s in range(S):
            gather_wait(s)
            transpose(s)
            store_start(s, s)
            gather_start(s, s + S)

        @pl.loop(S, CPW - S, step=S)
        def _(j0):
            for s in range(S):
                j = j0 + s
                gather_wait(s)
                store_wait(s)
                transpose(s)
                store_start(s, j)
                gather_start(s, j + S)

        for s in range(S):
            gather_wait(s)
            store_wait(s)
            transpose(s)
            store_start(s, CPW - S + s)
        for s in range(S):
            store_wait(s)

    return body


def kernel(x, table):
    batch, hist = x.shape
    vocab, d = table.shape
    idx = (
        x.T.astype(jnp.int32)
        .reshape(hist, batch // C, C)
        .reshape(NW, CPW, C)
    )
    out = _gather_kernel(d)(table, idx)
    return out.transpose(2, 4, 0, 1, 3).reshape(batch, hist, d)


# speedup vs baseline: 3.2223x; 1.3320x over previous
import functools

import jax
import jax.numpy as jnp
from jax import lax
from jax.experimental import pallas as pl
from jax.experimental.pallas import tpu as pltpu
from jax.experimental.pallas import tpu_sc as plsc

NC = 2
NS = 16
NW = NC * NS

V = 1000000
D = 64
TW = 128

LT_FULL = V // TW
LT_TAIL = V - LT_FULL * TW
K1_CH = 2
K1_R = K1_CH * TW
NCH = LT_FULL // K1_CH
K1_S = 2

C = 128
S = 4
L = 200
BB = 32
CPW = (L * BB) // NW


def _mesh():
    return plsc.VectorSubcoreMesh(
        core_axis_name="c", subcore_axis_name="s",
        num_cores=NC, num_subcores=NS)


def _relayout_kernel():
    @functools.partial(
        pl.kernel,
        out_type=jax.ShapeDtypeStruct((V // 8, 8, TW), jnp.float32),
        mesh=_mesh(),
        compiler_params=pltpu.CompilerParams(
            use_tc_tiling_on_sc=True, needs_layout_passes=False),
        scratch_types=[
            pltpu.VMEM((K1_S, D, K1_R), jnp.float32),
            pltpu.VMEM((K1_S, K1_R // 8, 8, TW), jnp.float32),
            pltpu.VMEM((D, LT_TAIL), jnp.float32),
            [pltpu.SemaphoreType.DMA] * K1_S,
            [pltpu.SemaphoreType.DMA] * K1_S,
        ],
    )
    def body(tblt_hbm, tail_hbm, out_hbm, slab_v, outv_v, tail_v, gsems, ssems):
        wid = lax.axis_index("s") * NC + lax.axis_index("c")
        nk = NCH // NW + jnp.where(wid < NCH % NW, 1, 0)

        iota = lax.iota(jnp.int32, 16)
        perms = [(iota + k) % 16 for k in range(16)]

        def in_start(s, ch):
            for dg in range(D // 8):
                for h in range(K1_CH):
                    pltpu.async_copy(
                        tblt_hbm.at[pl.ds(dg * 8, 8),
                                    pl.ds(ch * K1_R + h * TW, TW)],
                        slab_v.at[s, pl.ds(dg * 8, 8),
                                  pl.ds(h * TW, TW)],
                        gsems[s])

        def in_wait(s):
            pltpu.make_async_copy(
                tblt_hbm.at[:, pl.ds(0, K1_R)], slab_v.at[s],
                gsems[s]).wait()

        def out_start(s, ch):
            pltpu.async_copy(
                outv_v.at[s], out_hbm.at[pl.ds(ch * (K1_R // 8), K1_R // 8)],
                ssems[s])

        def out_wait(s):
            pltpu.make_async_copy(
                outv_v.at[s], out_hbm.at[pl.ds(0, K1_R // 8)],
                ssems[s]).wait()

        def transpose(src, s, nrow):
            @plsc.parallel_loop(0, nrow, step=16)
            def _(rr0):
                rvec = rr0 + iota
                gi = rvec // 8
                si = rvec % 8
                for c0 in range(0, D, 16):
                    for k in range(16):
                        dvec = c0 + perms[k]
                        val = plsc.load_gather(src, [dvec, rvec])
                        plsc.store_scatter(
                            outv_v.at[s], [gi, si, dvec], val)

        for s in range(K1_S):
            @pl.when(nk > s)
            def _():
                in_start(s, wid + s * NW)

        for s in range(K1_S):
            @pl.when(nk > s)
            def _():
                in_wait(s)
                transpose(slab_v.at[s], s, K1_R)
                out_start(s, wid + s * NW)

                @pl.when(nk > s + K1_S)
                def _():
                    in_start(s, wid + (s + K1_S) * NW)

        @pl.loop(K1_S, NCH // NW + 2, step=K1_S)
        def _(k):
            for s in range(K1_S):
                kk = k + s

                @pl.when(nk > kk)
                def _():
                    in_wait(s)
                    transpose(slab_v.at[s], s, K1_R)
                    out_wait(s)
                    out_start(s, wid + kk * NW)

                    @pl.when(nk > kk + K1_S)
                    def _():
                        in_start(s, wid + (kk + K1_S) * NW)

        for s in range(K1_S):
            @pl.when(nk > s)
            def _():
                out_wait(s)

        @pl.when(wid == NW - 1)
        def _():
            pltpu.sync_copy(tail_hbm, tail_v)
            transpose(tail_v, 0, LT_TAIL)
            pltpu.async_copy(
                outv_v.at[0, pl.ds(0, LT_TAIL // 8)],
                out_hbm.at[pl.ds((V - LT_TAIL) // 8, LT_TAIL // 8)],
                ssems[0])
            pltpu.make_async_copy(
                outv_v.at[0, pl.ds(0, LT_TAIL // 8)],
                out_hbm.at[pl.ds(0, LT_TAIL // 8)], ssems[0]).wait()

    return body


def _gather_kernel():
    dg = D // 8

    @functools.partial(
        pl.kernel,
        out_type=jax.ShapeDtypeStruct((L, dg, BB, 8, C), jnp.float32),
        mesh=_mesh(),
        compiler_params=pltpu.CompilerParams(
            use_tc_tiling_on_sc=False, needs_layout_passes=False),
        scratch_types=[
            pltpu.VMEM((CPW, C), jnp.int32),
            pltpu.VMEM((S, C, TW), jnp.float32),
            pltpu.VMEM((S, dg, 8, C + 1), jnp.float32),
            [pltpu.SemaphoreType.DMA] * S,
            [pltpu.SemaphoreType.DMA] * S,
        ],
    )
    def body(table_hbm, idx_hbm, out_hbm, idx_v, rows_v, rowst_v, gsems, ssems):
        wid = lax.axis_index("s") * NC + lax.axis_index("c")
        cbase = wid * CPW

        pltpu.sync_copy(idx_hbm.at[wid], idx_v)

        dvecs = [g * 16 + lax.iota(jnp.int32, 16) for g in range(D // 16)]
        divecs = [v // 8 for v in dvecs]
        djvecs = [v % 8 for v in dvecs]

        def gather_start(s, j):
            pltpu.async_copy(table_hbm.at[idx_v.at[j]], rows_v.at[s], gsems[s])

        def gather_wait(s):
            pltpu.make_async_copy(
                table_hbm.at[idx_v.at[0]], rows_v.at[s], gsems[s]).wait()

        def _st_src(s):
            return rowst_v.at[s, :, :, pl.ds(0, C)]

        def store_start(s, j):
            c = cbase + j
            pltpu.async_copy(
                _st_src(s), out_hbm.at[c // BB, :, c % BB], ssems[s])

        def store_wait(s):
            pltpu.make_async_copy(
                _st_src(s), out_hbm.at[0, :, 0], ssems[s]).wait()

        def transpose(s):
            @plsc.parallel_loop(0, C, unroll=2)
            def _(b):
                bvec = jnp.full((16,), b, dtype=jnp.int32)
                for g in range(D // 16):
                    val = rows_v[s, b, pl.ds(g * 16, 16)]
                    plsc.store_scatter(
                        rowst_v.at[s], [divecs[g], djvecs[g], bvec], val)

        for s in range(S):
            gather_start(s, s)

        for s in range(S):
            gather_wait(s)
            transpose(s)
            store_start(s, s)
            gather_start(s, s + S)

        @pl.loop(S, CPW - S, step=S)
        def _(j0):
            for s in range(S):
                j = j0 + s
                gather_wait(s)
                store_wait(s)
                transpose(s)
                store_start(s, j)
                gather_start(s, j + S)

        for s in range(S):
            gather_wait(s)
            store_wait(s)
            transpose(s)
            store_start(s, CPW - S + s)
        for s in range(S):
            store_wait(s)

    return body


def kernel(x, table):
    batch, hist = x.shape
    idx = (
        x.T.astype(jnp.int32)
        .reshape(hist, batch // C, C)
        .reshape(NW, CPW, C)
    )
    padded = _relayout_kernel()(table.T, table[V - LT_TAIL:].T)
    tbl_lin = padded.reshape(V, TW)
    out = _gather_kernel()(tbl_lin, idx)
    return out.transpose(2, 4, 0, 1, 3).reshape(batch, hist, D)


# speedup vs baseline: 3.5768x; 1.1100x over previous
import functools

import jax
import jax.numpy as jnp
from jax import lax
from jax.experimental import pallas as pl
from jax.experimental.pallas import tpu as pltpu
from jax.experimental.pallas import tpu_sc as plsc

NC = 2
NS = 16
NW = NC * NS

V = 1000000
D = 64
TW = 128

LT_FULL = V // TW
LT_TAIL = V - LT_FULL * TW
K1_CH = 2
K1_R = K1_CH * TW
NCH = LT_FULL // K1_CH
K1_S = 2

C = 128
S = 4
L = 200
BB = 32
CPW = (L * BB) // NW


def _mesh():
    return plsc.VectorSubcoreMesh(
        core_axis_name="c", subcore_axis_name="s",
        num_cores=NC, num_subcores=NS)


def _relayout_kernel():
    @functools.partial(
        pl.kernel,
        out_type=jax.ShapeDtypeStruct((V * D // 1024, 8, 128), jnp.float32),
        mesh=_mesh(),
        compiler_params=pltpu.CompilerParams(
            use_tc_tiling_on_sc=True, needs_layout_passes=False),
        scratch_types=[
            pltpu.VMEM((K1_S, D, K1_R), jnp.float32),
            pltpu.VMEM((K1_S, K1_R * D // 1024, 8, 128), jnp.float32),
            pltpu.VMEM((D, LT_TAIL), jnp.float32),
            [pltpu.SemaphoreType.DMA] * K1_S,
            [pltpu.SemaphoreType.DMA] * K1_S,
        ],
    )
    def body(tblt_hbm, tail_hbm, out_hbm, slab_v, outv_v, tail_v, gsems, ssems):
        wid = lax.axis_index("s") * NC + lax.axis_index("c")
        nk = NCH // NW + jnp.where(wid < NCH % NW, 1, 0)

        iota = lax.iota(jnp.int32, 16)
        perms = [(iota + k) % 16 for k in range(16)]

        def in_start(s, ch):
            for dg in range(D // 8):
                for h in range(K1_CH):
                    pltpu.async_copy(
                        tblt_hbm.at[pl.ds(dg * 8, 8),
                                    pl.ds(ch * K1_R + h * TW, TW)],
                        slab_v.at[s, pl.ds(dg * 8, 8),
                                  pl.ds(h * TW, TW)],
                        gsems[s])

        def in_wait(s):
            pltpu.make_async_copy(
                tblt_hbm.at[:, pl.ds(0, K1_R)], slab_v.at[s],
                gsems[s]).wait()

        ngr = K1_R * D // 1024

        def out_start(s, ch):
            pltpu.async_copy(
                outv_v.at[s], out_hbm.at[pl.ds(ch * ngr, ngr)], ssems[s])

        def out_wait(s):
            pltpu.make_async_copy(
                outv_v.at[s], out_hbm.at[pl.ds(0, ngr)], ssems[s]).wait()

        def transpose(src, s, nrow):
            @plsc.parallel_loop(0, nrow, step=16)
            def _(rr0):
                rvec = rr0 + iota
                gi = rvec // 16
                si = (rvec // 2) % 8
                half = (rvec % 2) * D
                for c0 in range(0, D, 16):
                    for k in range(16):
                        dvec = c0 + perms[k]
                        val = plsc.load_gather(src, [dvec, rvec])
                        plsc.store_scatter(
                            outv_v.at[s], [gi, si, half + dvec], val)

        for s in range(K1_S):
            @pl.when(nk > s)
            def _():
                in_start(s, wid + s * NW)

        for s in range(K1_S):
            @pl.when(nk > s)
            def _():
                in_wait(s)
                transpose(slab_v.at[s], s, K1_R)
                out_start(s, wid + s * NW)

                @pl.when(nk > s + K1_S)
                def _():
                    in_start(s, wid + (s + K1_S) * NW)

        @pl.loop(K1_S, NCH // NW + 2, step=K1_S)
        def _(k):
            for s in range(K1_S):
                kk = k + s

                @pl.when(nk > kk)
                def _():
                    in_wait(s)
                    transpose(slab_v.at[s], s, K1_R)
                    out_wait(s)
                    out_start(s, wid + kk * NW)

                    @pl.when(nk > kk + K1_S)
                    def _():
                        in_start(s, wid + (kk + K1_S) * NW)

        for s in range(K1_S):
            @pl.when(nk > s)
            def _():
                out_wait(s)

        @pl.when(wid == NW - 1)
        def _():
            pltpu.sync_copy(tail_hbm, tail_v)
            transpose(tail_v, 0, LT_TAIL)
            tgr = LT_TAIL * D // 1024
            pltpu.async_copy(
                outv_v.at[0, pl.ds(0, tgr)],
                out_hbm.at[pl.ds((V - LT_TAIL) * D // 1024, tgr)],
                ssems[0])
            pltpu.make_async_copy(
                outv_v.at[0, pl.ds(0, tgr)],
                out_hbm.at[pl.ds(0, tgr)], ssems[0]).wait()

    return body


def _gather_kernel():
    dg = D // 8

    @functools.partial(
        pl.kernel,
        out_type=jax.ShapeDtypeStruct((L, dg, BB, 8, C), jnp.float32),
        mesh=_mesh(),
        compiler_params=pltpu.CompilerParams(
            use_tc_tiling_on_sc=False, needs_layout_passes=False),
        scratch_types=[
            pltpu.VMEM((CPW, C), jnp.int32),
            pltpu.VMEM((S, C, D), jnp.float32),
            pltpu.VMEM((S, dg, 8, C + 1), jnp.float32),
            [pltpu.SemaphoreType.DMA] * S,
            [pltpu.SemaphoreType.DMA] * S,
        ],
    )
    def body(table_hbm, idx_hbm, out_hbm, idx_v, rows_v, rowst_v, gsems, ssems):
        wid = lax.axis_index("s") * NC + lax.axis_index("c")
        cbase = wid * CPW

        pltpu.sync_copy(idx_hbm.at[wid], idx_v)

        dvecs = [g * 16 + lax.iota(jnp.int32, 16) for g in range(D // 16)]
        divecs = [v // 8 for v in dvecs]
        djvecs = [v % 8 for v in dvecs]

        def gather_start(s, j):
            pltpu.async_copy(table_hbm.at[idx_v.at[j]], rows_v.at[s], gsems[s])

        def gather_wait(s):
            pltpu.make_async_copy(
                table_hbm.at[idx_v.at[0]], rows_v.at[s], gsems[s]).wait()

        def _st_src(s):
            return rowst_v.at[s, :, :, pl.ds(0, C)]

        def store_start(s, j):
            c = cbase + j
            pltpu.async_copy(
                _st_src(s), out_hbm.at[c // BB, :, c % BB], ssems[s])

        def store_wait(s):
            pltpu.make_async_copy(
                _st_src(s), out_hbm.at[0, :, 0], ssems[s]).wait()

        def transpose(s):
            @plsc.parallel_loop(0, C, unroll=2)
            def _(b):
                bvec = jnp.full((16,), b, dtype=jnp.int32)
                for g in range(D // 16):
                    val = rows_v[s, b, pl.ds(g * 16, 16)]
                    plsc.store_scatter(
                        rowst_v.at[s], [divecs[g], djvecs[g], bvec], val)

        for s in range(S):
            gather_start(s, s)

        for s in range(S):
            gather_wait(s)
            transpose(s)
            store_start(s, s)
            gather_start(s, s + S)

        @pl.loop(S, CPW - S, step=S)
        def _(j0):
            for s in range(S):
                j = j0 + s
                gather_wait(s)
                store_wait(s)
                transpose(s)
                store_start(s, j)
                gather_start(s, j + S)

        for s in range(S):
            gather_wait(s)
            store_wait(s)
            transpose(s)
            store_start(s, CPW - S + s)
        for s in range(S):
            store_wait(s)

    return body


def kernel(x, table):
    batch, hist = x.shape
    idx = (
        x.T.astype(jnp.int32)
        .reshape(hist, batch // C, C)
        .reshape(NW, CPW, C)
    )
    padded = _relayout_kernel()(table.T, table[V - LT_TAIL:].T)
    tbl_lin = padded.reshape(V, D)
    out = _gather_kernel()(tbl_lin, idx)
    return out.transpose(2, 4, 0, 1, 3).reshape(batch, hist, D)
